# baseline (speedup 1.0000x reference)
import functools

import jax
import jax.numpy as jnp
from jax import lax
from jax.experimental import pallas as pl
from jax.experimental.pallas import tpu as pltpu
from jax.experimental.pallas import tpu_sc as plsc

N_NODES_ = 10000
N_EDGES_ = 320000
D_ = 128
NW_ = 32
PER_TILE_ = N_EDGES_ // NW_
C_ = 96
NCHUNK_ = 105
NSLOT_ = 3
PAD_TILE_ = NCHUNK_ * C_
FULL_CHUNKS_ = PER_TILE_ // C_
TAIL_ = PER_TILE_ - FULL_CHUNKS_ * C_
EPS_ = 1e-8


def _norm_body(h_ref, n_ref):
    x = h_ref[...]
    n_ref[...] = jnp.sqrt(jnp.sum(x * x, axis=1))


def _node_norms(h):
    return pl.pallas_call(
        _norm_body,
        out_shape=jax.ShapeDtypeStruct((N_NODES_,), jnp.float32),
    )(h)


def _sc_body(h_hbm, src_hbm, dst_hbm, n_hbm, out_hbm,
             ntab, hsh, buf0, buf1, buf2, ob0, ob1, ob2,
             ixs0, ixs1, ixs2, ixd0, ixd1, ixd2,
             sb0, sb1, sb2, sa0, sa1, sa2, so0, so1, so2,
             si0, si1, si2):
    cid = lax.axis_index("c")
    sid = lax.axis_index("s")
    wid = sid * 2 + cid
    base = wid * PER_TILE_

    rows_per_sub = 624
    rbase = sid * rows_per_sub
    pltpu.sync_copy(h_hbm.at[pl.ds(rbase, rows_per_sub)],
                    hsh.at[pl.ds(rbase, rows_per_sub)])

    @pl.when(sid == 0)
    def _():
        pltpu.sync_copy(h_hbm.at[pl.ds(9984, 16)], hsh.at[pl.ds(9984, 16)])

    pltpu.sync_copy(n_hbm, ntab)

    lane = lax.broadcasted_iota(jnp.int32, (16,), 0)
    bufs = (buf0, buf1, buf2)
    obufs = (ob0, ob1, ob2)
    ixs = (ixs0, ixs1, ixs2)
    ixd = (ixd0, ixd1, ixd2)
    semb = (sb0, sb1, sb2)
    sema = (sa0, sa1, sa2)
    semo = (so0, so1, so2)
    semi = (si0, si1, si2)

    zeros_i = jnp.zeros((16,), jnp.int32)

    def fire_idx(k, slot):
        pltpu.async_copy(src_hbm.at[pl.ds(base + k * C_, C_)],
                         ixs[slot], semi[slot])
        pltpu.async_copy(dst_hbm.at[pl.ds(base + k * C_, C_)],
                         ixd[slot], semi[slot])

    def fire_idx_tail(slot):
        pltpu.async_copy(src_hbm.at[pl.ds(base + FULL_CHUNKS_ * C_, TAIL_)],
                         ixs[slot].at[pl.ds(0, TAIL_)], semi[slot])
        pltpu.async_copy(dst_hbm.at[pl.ds(base + FULL_CHUNKS_ * C_, TAIL_)],
                         ixd[slot].at[pl.ds(0, TAIL_)], semi[slot])

    def wait_idx(slot, tail=False):
        m = TAIL_ if tail else C_
        pltpu.make_async_copy(src_hbm.at[pl.ds(0, m)],
                              ixs[slot].at[pl.ds(0, m)], semi[slot]).wait()
        pltpu.make_async_copy(src_hbm.at[pl.ds(0, m)],
                              ixd[slot].at[pl.ds(0, m)], semi[slot]).wait()

    def fire_base(slot):
        pltpu.async_copy(hsh.at[ixs[slot]], bufs[slot], semb[slot])

    def fire_add(slot):
        pltpu.async_copy(hsh.at[ixd[slot]], bufs[slot], sema[slot], add=True)

    def wait_base(slot):
        pltpu.make_async_copy(h_hbm.at[pl.ds(0, C_)], bufs[slot],
                              semb[slot]).wait()

    def wait_add(slot):
        pltpu.make_async_copy(h_hbm.at[pl.ds(0, C_)], bufs[slot],
                              sema[slot]).wait()

    def fire_out(k, slot):
        pltpu.async_copy(obufs[slot], out_hbm.at[pl.ds(base + k * C_, C_)],
                         semo[slot])

    def wait_out(slot):
        pltpu.make_async_copy(obufs[slot], out_hbm.at[pl.ds(0, C_)],
                              semo[slot]).wait()

    def compute(slot):
        buf = bufs[slot]
        ob = obufs[slot]

        def group_body(g, _):
            ssq16 = jnp.zeros((16,), jnp.float32)
            for e16 in range(16):
                ec = g * 16 + e16
                v = buf[ec, pl.ds(0, 16)]
                acc = v * v
                for j in range(1, 8):
                    v = buf[ec, pl.ds(16 * j, 16)]
                    acc = acc + v * v
                for sh in (8, 4, 2, 1):
                    acc = acc + acc.at[lane ^ sh].get(mode="promise_in_bounds")
                ssq16 = jnp.where(lane == e16, acc, ssq16)
            goff = g * 16
            is16 = ixs[slot][pl.ds(goff, 16)]
            id16 = ixd[slot][pl.ds(goff, 16)]
            ns = plsc.load_gather(ntab, [is16])
            nd = plsc.load_gather(ntab, [id16])
            dot16 = (ssq16 - ns * ns - nd * nd) * 0.5
            ob[pl.ds(goff, 16)] = dot16 / jnp.maximum(ns * nd, EPS_)
            return 0

        lax.fori_loop(0, C_ // 16, group_body, 0)

    plsc.subcore_barrier()

    fire_idx(0, 0)
    fire_idx(1, 1)
    fire_idx(2, 2)
    wait_idx(0)
    fire_base(0)
    wait_idx(1)
    fire_base(1)
    wait_base(0)
    fire_add(0)

    def ring_body(i, _):
        k0 = i * NSLOT_
        for b in range(NSLOT_):
            k = k0 + b
            nslot = (b + 1) % NSLOT_

            @pl.when(k + 1 < NCHUNK_)
            def _():
                wait_base(nslot)
                fire_add(nslot)

            @pl.when(k >= NSLOT_)
            def _():
                wait_out(b)

            wait_add(b)
            compute(b)

            @pl.when(k < FULL_CHUNKS_)
            def _():
                fire_out(k, b)

            @pl.when(k + NSLOT_ < FULL_CHUNKS_)
            def _():
                fire_idx(k + NSLOT_, b)

            @pl.when(k + NSLOT_ == FULL_CHUNKS_)
            def _():
                for t in range(TAIL_ // 16, C_ // 16):
                    ixs[b][pl.ds(16 * t, 16)] = zeros_i
                    ixd[b][pl.ds(16 * t, 16)] = zeros_i
                fire_idx_tail(b)

            @pl.when(jnp.logical_and(k + 2 < NCHUNK_, k + 2 != FULL_CHUNKS_))
            def _():
                wait_idx((b + 2) % NSLOT_, tail=False)
                fire_base((b + 2) % NSLOT_)

            @pl.when(k + 2 == FULL_CHUNKS_)
            def _():
                wait_idx((b + 2) % NSLOT_, tail=True)
                fire_base((b + 2) % NSLOT_)
        return 0

    lax.fori_loop(0, NCHUNK_ // NSLOT_, ring_body, 0)

    pltpu.sync_copy(obufs[FULL_CHUNKS_ % NSLOT_].at[pl.ds(0, TAIL_)],
                    out_hbm.at[pl.ds(base + FULL_CHUNKS_ * C_, TAIL_)])

    wait_out((FULL_CHUNKS_ - 2) % NSLOT_)
    wait_out((FULL_CHUNKS_ - 1) % NSLOT_)


def _edge_cos_sc(h, src, dst, n):
    mesh = plsc.VectorSubcoreMesh(core_axis_name="c", subcore_axis_name="s")
    f = functools.partial(
        pl.kernel,
        mesh=mesh,
        out_type=jax.ShapeDtypeStruct((N_EDGES_,), jnp.float32),
        scratch_types=[
            pltpu.VMEM((N_NODES_,), jnp.float32),
            pltpu.VMEM_SHARED((N_NODES_, D_), jnp.float32),
            pltpu.VMEM((C_, D_), jnp.float32),
            pltpu.VMEM((C_, D_), jnp.float32),
            pltpu.VMEM((C_, D_), jnp.float32),
            pltpu.VMEM((C_,), jnp.float32),
            pltpu.VMEM((C_,), jnp.float32),
            pltpu.VMEM((C_,), jnp.float32),
            pltpu.VMEM((C_,), jnp.int32),
            pltpu.VMEM((C_,), jnp.int32),
            pltpu.VMEM((C_,), jnp.int32),
            pltpu.VMEM((C_,), jnp.int32),
            pltpu.VMEM((C_,), jnp.int32),
            pltpu.VMEM((C_,), jnp.int32),
        ] + [pltpu.SemaphoreType.DMA] * 12,
        compiler_params=pltpu.CompilerParams(needs_layout_passes=False),
    )(_sc_body)
    return f(h, src, dst, n)


def kernel(h, edge_index):
    ei = edge_index.astype(jnp.int32)
    n = _node_norms(h)
    return _edge_cos_sc(h, ei[0], ei[1], n)


# speedup vs baseline: 1.1547x; 1.1547x over previous
import functools

import jax
import jax.numpy as jnp
from jax import lax
from jax.experimental import pallas as pl
from jax.experimental.pallas import tpu as pltpu
from jax.experimental.pallas import tpu_sc as plsc

N_NODES_ = 10000
N_EDGES_ = 320000
D_ = 128
NW_ = 32
PER_TILE_ = N_EDGES_ // NW_
C_ = 48
NCHUNK_ = 210
NSLOT_ = 3
PAD_TILE_ = NCHUNK_ * C_
FULL_CHUNKS_ = PER_TILE_ // C_
TAIL_ = PER_TILE_ - FULL_CHUNKS_ * C_
EPS_ = 1e-8


def _norm_body(h_ref, n_ref):
    x = h_ref[...]
    n_ref[...] = jnp.sqrt(jnp.sum(x * x, axis=1))


def _node_norms(h):
    return pl.pallas_call(
        _norm_body,
        out_shape=jax.ShapeDtypeStruct((N_NODES_,), jnp.float32),
    )(h)


def _sc_body(h_hbm, src_hbm, dst_hbm, n_hbm, out_hbm,
             idx_s, idx_d, ntab, hsh, buf0, buf1, buf2, ob0, ob1, ob2,
             sb0, sb1, sb2, sa0, sa1, sa2, so0, so1, so2):
    cid = lax.axis_index("c")
    sid = lax.axis_index("s")
    wid = sid * 2 + cid
    base = wid * PER_TILE_

    rows_per_sub = 624
    rbase = sid * rows_per_sub
    pltpu.sync_copy(h_hbm.at[pl.ds(rbase, rows_per_sub)],
                    hsh.at[pl.ds(rbase, rows_per_sub)])

    @pl.when(sid == 0)
    def _():
        pltpu.sync_copy(h_hbm.at[pl.ds(9984, 16)], hsh.at[pl.ds(9984, 16)])

    pltpu.sync_copy(src_hbm.at[pl.ds(base, PER_TILE_)], idx_s.at[pl.ds(0, PER_TILE_)])
    pltpu.sync_copy(dst_hbm.at[pl.ds(base, PER_TILE_)], idx_d.at[pl.ds(0, PER_TILE_)])
    pltpu.sync_copy(n_hbm, ntab)

    zeros_i = jnp.zeros((16,), jnp.int32)
    for t in range((PAD_TILE_ - PER_TILE_) // 16):
        idx_s[pl.ds(PER_TILE_ + 16 * t, 16)] = zeros_i
        idx_d[pl.ds(PER_TILE_ + 16 * t, 16)] = zeros_i

    lane = lax.broadcasted_iota(jnp.int32, (16,), 0)
    bufs = (buf0, buf1, buf2)
    obufs = (ob0, ob1, ob2)
    semb = (sb0, sb1, sb2)
    sema = (sa0, sa1, sa2)
    semo = (so0, so1, so2)

    def fire_base(k, slot):
        pltpu.async_copy(hsh.at[idx_s.at[pl.ds(k * C_, C_)]],
                         bufs[slot], semb[slot])

    def fire_add(k, slot):
        pltpu.async_copy(hsh.at[idx_d.at[pl.ds(k * C_, C_)]],
                         bufs[slot], sema[slot], add=True)

    def wait_base(slot):
        pltpu.make_async_copy(h_hbm.at[pl.ds(0, C_)], bufs[slot],
                              semb[slot]).wait()

    def wait_add(slot):
        pltpu.make_async_copy(h_hbm.at[pl.ds(0, C_)], bufs[slot],
                              sema[slot]).wait()

    def fire_out(k, slot):
        pltpu.async_copy(obufs[slot], out_hbm.at[pl.ds(base + k * C_, C_)],
                         semo[slot])

    def wait_out(slot):
        pltpu.make_async_copy(obufs[slot], out_hbm.at[pl.ds(0, C_)],
                              semo[slot]).wait()

    def compute(k, slot):
        buf = bufs[slot]
        ob = obufs[slot]
        for g in range(C_ // 16):
            ssq16 = jnp.zeros((16,), jnp.float32)
            for e16 in range(16):
                ec = g * 16 + e16
                v = buf[ec, pl.ds(0, 16)]
                acc = v * v
                for j in range(1, 8):
                    v = buf[ec, pl.ds(16 * j, 16)]
                    acc = acc + v * v
                for sh in (8, 4, 2, 1):
                    acc = acc + acc.at[lane ^ sh].get(mode="promise_in_bounds")
                ssq16 = jnp.where(lane == e16, acc, ssq16)
            eoff = k * C_ + g * 16
            is16 = idx_s[pl.ds(eoff, 16)]
            id16 = idx_d[pl.ds(eoff, 16)]
            ns = plsc.load_gather(ntab, [is16])
            nd = plsc.load_gather(ntab, [id16])
            dot16 = (ssq16 - ns * ns - nd * nd) * 0.5
            ob[pl.ds(g * 16, 16)] = dot16 / jnp.maximum(ns * nd, EPS_)

    plsc.subcore_barrier()

    for p in range(NSLOT_):
        fire_base(p, p)
    wait_base(0)
    fire_add(0, 0)

    def ring_body(i, _):
        k0 = i * NSLOT_
        for b in range(NSLOT_):
            k = k0 + b
            nslot = (b + 1) % NSLOT_

            @pl.when(k + 1 < NCHUNK_)
            def _():
                wait_base(nslot)
                fire_add(k + 1, nslot)

            @pl.when(k >= NSLOT_)
            def _():
                wait_out(b)

            wait_add(b)
            compute(k, b)

            @pl.when(k < FULL_CHUNKS_)
            def _():
                fire_out(k, b)

            @pl.when(k + NSLOT_ < NCHUNK_)
            def _():
                fire_base(k + NSLOT_, b)
        return 0

    lax.fori_loop(0, NCHUNK_ // NSLOT_, ring_body, 0)

    pltpu.sync_copy(obufs[FULL_CHUNKS_ % NSLOT_].at[pl.ds(0, TAIL_)],
                    out_hbm.at[pl.ds(base + FULL_CHUNKS_ * C_, TAIL_)])

    wait_out((FULL_CHUNKS_ - 1) % NSLOT_)


def _edge_cos_sc(h, src, dst, n):
    mesh = plsc.VectorSubcoreMesh(core_axis_name="c", subcore_axis_name="s")
    f = functools.partial(
        pl.kernel,
        mesh=mesh,
        out_type=jax.ShapeDtypeStruct((N_EDGES_,), jnp.float32),
        scratch_types=[
            pltpu.VMEM((PAD_TILE_,), jnp.int32),
            pltpu.VMEM((PAD_TILE_,), jnp.int32),
            pltpu.VMEM((N_NODES_,), jnp.float32),
            pltpu.VMEM_SHARED((N_NODES_, D_), jnp.float32),
            pltpu.VMEM((C_, D_), jnp.float32),
            pltpu.VMEM((C_, D_), jnp.float32),
            pltpu.VMEM((C_, D_), jnp.float32),
            pltpu.VMEM((C_,), jnp.float32),
            pltpu.VMEM((C_,), jnp.float32),
            pltpu.VMEM((C_,), jnp.float32),
        ] + [pltpu.SemaphoreType.DMA] * 9,
        compiler_params=pltpu.CompilerParams(needs_layout_passes=False),
    )(_sc_body)
    return f(h, src, dst, n)


def kernel(h, edge_index):
    ei = edge_index.astype(jnp.int32)
    n = _node_norms(h)
    return _edge_cos_sc(h, ei[0], ei[1], n)
